# Initial kernel scaffold; baseline (speedup 1.0000x reference)
import jax
import jax.numpy as jnp
from jax.experimental import pallas as pl


def kernel(x_nchw, w1, b1, w2, b2, w_fc1, b_fc1, w_fc2, b_fc2):
    raise NotImplementedError("write your pallas kernel here")


# speedup vs baseline: 3.0223x; 3.0223x over previous
import functools

import jax
import jax.numpy as jnp
from jax.experimental import pallas as pl
from jax.experimental.pallas import tpu as pltpu


def _fused_cnn_kernel(bt, x_ref, w1t_ref, b1t_ref, w2t_ref, b2t_ref,
                      wf1_ref, bf1_ref, wf2_ref, bf2_ref, o_ref):
    f32 = jnp.float32
    x = x_ref[...]

    pooled_halves = []
    for hw in range(2):
        base = 48 * hw
        cols = [x[:, dh:dh + 32, base:base + 54] for dh in range(3)]
        p = jnp.concatenate(cols, axis=-1).reshape(bt * 32, 162)
        a = jnp.dot(p, w1t_ref[...], preferred_element_type=f32)
        a = jnp.maximum(a + b1t_ref[...], 0.0)
        a = a.reshape(bt, 16, 2, 512).max(axis=2)
        a = a.reshape(bt, 16, 8, 2, 32).max(axis=3)
        pooled_halves.append(a.reshape(bt, 16, 256))
    a1 = jnp.concatenate(pooled_halves, axis=-1).astype(jnp.bfloat16)

    zc = jnp.zeros((bt, 16, 32), jnp.bfloat16)
    s2 = jnp.concatenate([zc, a1, zc], axis=-1)
    zr = jnp.zeros((bt, 1, 576), jnp.bfloat16)
    s2 = jnp.concatenate([zr, s2, zr], axis=1)

    quarters = []
    for q in range(4):
        lo = 128 * q
        acc = None
        for dh in range(3):
            lhs = s2[:, dh:dh + 16, lo:lo + 192].reshape(bt * 16, 192)
            d = jnp.dot(lhs, w2t_ref[dh], preferred_element_type=f32)
            acc = d if acc is None else acc + d
        acc = jnp.maximum(acc + b2t_ref[...], 0.0)
        acc = acc.reshape(bt, 8, 2, 256).max(axis=2)
        acc = acc.reshape(bt, 8, 2, 2, 64).max(axis=3)
        quarters.append(acc.reshape(bt, 8, 128))
    c2 = jnp.concatenate(quarters, axis=-1).astype(jnp.bfloat16)

    xf = c2.reshape(bt, 4096)
    h1 = jnp.dot(xf, wf1_ref[...], preferred_element_type=f32)
    h1 = jnp.maximum(h1 + bf1_ref[...], 0.0).astype(jnp.bfloat16)
    z = jnp.dot(h1, wf2_ref[...], preferred_element_type=f32) + bf2_ref[...]
    z = z - jnp.max(z, axis=-1, keepdims=True)
    e = jnp.exp(z)
    o_ref[...] = (e / jnp.sum(e, axis=-1, keepdims=True)).astype(o_ref.dtype)


def kernel(x_nchw, w1, b1, w2, b2, w_fc1, b_fc1, w_fc2, b_fc2):
    B = x_nchw.shape[0]
    bt = 128 if B % 128 == 0 else B

    x = jnp.transpose(x_nchw, (0, 2, 3, 1))
    x = jnp.pad(x, ((0, 0), (1, 1), (1, 1), (0, 0)))
    xr = x.reshape(B, 34, 102).astype(jnp.bfloat16)

    w1r = w1.reshape(3, 3, 3, 32)
    E1 = (jnp.arange(18)[None, :, None]
          == jnp.arange(16)[None, None, :] + jnp.arange(3)[:, None, None])
    w1t = jnp.einsum('kdw,hkco->hdcwo', E1.astype(w1.dtype), w1r)
    w1t = w1t.reshape(162, 512).astype(jnp.bfloat16)

    w2r = w2.reshape(3, 3, 32, 64)
    E2 = (jnp.arange(6)[None, :, None]
          == jnp.arange(4)[None, None, :] + jnp.arange(3)[:, None, None])
    w2t = jnp.einsum('kdw,hkco->hdcwo', E2.astype(w2.dtype), w2r)
    w2t = w2t.reshape(3, 192, 256).astype(jnp.bfloat16)

    b1t = jnp.tile(b1, (1, 16))
    b2t = jnp.tile(b2, (1, 4))
    wf1 = w_fc1.astype(jnp.bfloat16)
    wf2 = w_fc2.astype(jnp.bfloat16)

    kernel_fn = functools.partial(_fused_cnn_kernel, bt)
    return pl.pallas_call(
        kernel_fn,
        out_shape=jax.ShapeDtypeStruct((B, 100), jnp.float32),
        grid=(B // bt,),
        in_specs=[
            pl.BlockSpec((bt, 34, 102), lambda i: (i, 0, 0)),
            pl.BlockSpec((162, 512), lambda i: (0, 0)),
            pl.BlockSpec((1, 512), lambda i: (0, 0)),
            pl.BlockSpec((3, 192, 256), lambda i: (0, 0, 0)),
            pl.BlockSpec((1, 256), lambda i: (0, 0)),
            pl.BlockSpec((4096, 512), lambda i: (0, 0)),
            pl.BlockSpec((1, 512), lambda i: (0, 0)),
            pl.BlockSpec((512, 100), lambda i: (0, 0)),
            pl.BlockSpec((1, 100), lambda i: (0, 0)),
        ],
        out_specs=pl.BlockSpec((bt, 100), lambda i: (i, 0)),
        compiler_params=pltpu.CompilerParams(
            dimension_semantics=("parallel",),
            vmem_limit_bytes=64 * 1024 * 1024),
    )(xr, w1t, b1t, w2t, b2t, wf1, b_fc1, wf2, b_fc2)


# speedup vs baseline: 7.0900x; 2.3459x over previous
import functools

import jax
import jax.numpy as jnp
from jax.experimental import pallas as pl
from jax.experimental.pallas import tpu as pltpu


def _fused_cnn_kernel(bt, x_ref, w1t_ref, b1t_ref, w2t_ref, b2t_ref,
                      wf1_ref, bf1_ref, wf2_ref, bf2_ref, o_ref):
    f32 = jnp.float32
    x = x_ref[...]

    acc = None
    for dh in range(3):
        lhs = x[:, dh:dh + 32, :].reshape(bt * 32, 102)
        d = jnp.dot(lhs, w1t_ref[dh], preferred_element_type=f32)
        acc = d if acc is None else acc + d
    acc = jnp.maximum(acc + b1t_ref[...], 0.0)
    acc = jnp.maximum(acc[:, :512], acc[:, 512:])
    acc = acc.reshape(bt, 16, 2, 512).max(axis=2)
    a1 = acc.astype(jnp.bfloat16)

    zc = jnp.zeros((bt, 16, 32), jnp.bfloat16)
    s2 = jnp.concatenate([zc, a1, zc], axis=-1)
    zr = jnp.zeros((bt, 1, 576), jnp.bfloat16)
    s2 = jnp.concatenate([zr, s2, zr], axis=1)

    acc = None
    for dh in range(3):
        lhs = s2[:, dh:dh + 16, :].reshape(bt * 16, 576)
        d = jnp.dot(lhs, w2t_ref[dh], preferred_element_type=f32)
        acc = d if acc is None else acc + d
    acc = jnp.maximum(acc + b2t_ref[...], 0.0)
    acc = jnp.maximum(acc[:, :512], acc[:, 512:])
    acc = acc.reshape(bt, 8, 2, 512).max(axis=2)
    c2 = acc.astype(jnp.bfloat16)

    xf = c2.reshape(bt, 4096)
    h1 = jnp.dot(xf, wf1_ref[...], preferred_element_type=f32)
    h1 = jnp.maximum(h1 + bf1_ref[...], 0.0).astype(jnp.bfloat16)
    z = jnp.dot(h1, wf2_ref[...], preferred_element_type=f32) + bf2_ref[...]
    z = z - jnp.max(z, axis=-1, keepdims=True)
    e = jnp.exp(z)
    o_ref[...] = (e / jnp.sum(e, axis=-1, keepdims=True)).astype(o_ref.dtype)


def kernel(x_nchw, w1, b1, w2, b2, w_fc1, b_fc1, w_fc2, b_fc2):
    B = x_nchw.shape[0]
    bt = 128 if B % 128 == 0 else B

    x = jnp.transpose(x_nchw, (0, 2, 3, 1))
    x = jnp.pad(x, ((0, 0), (1, 1), (1, 1), (0, 0)))
    xr = x.reshape(B, 34, 102).astype(jnp.bfloat16)

    w1r = w1.reshape(3, 3, 3, 32)
    E1 = (jnp.arange(34)[None, :, None, None]
          == 2 * jnp.arange(16)[None, None, None, :]
          + jnp.arange(2)[None, None, :, None]
          + jnp.arange(3)[:, None, None, None])
    w1t = jnp.einsum('kdpw,hkco->hdcpwo', E1.astype(w1.dtype), w1r)
    w1t = w1t.reshape(3, 102, 1024).astype(jnp.bfloat16)

    w2r = w2.reshape(3, 3, 32, 64)
    E2 = (jnp.arange(18)[None, :, None, None]
          == 2 * jnp.arange(8)[None, None, None, :]
          + jnp.arange(2)[None, None, :, None]
          + jnp.arange(3)[:, None, None, None])
    w2t = jnp.einsum('kdpw,hkco->hdcpwo', E2.astype(w2.dtype), w2r)
    w2t = w2t.reshape(3, 576, 1024).astype(jnp.bfloat16)

    b1t = jnp.tile(b1, (1, 32))
    b2t = jnp.tile(b2, (1, 16))
    wf1 = w_fc1.astype(jnp.bfloat16)
    wf2 = w_fc2.astype(jnp.bfloat16)

    kernel_fn = functools.partial(_fused_cnn_kernel, bt)
    return pl.pallas_call(
        kernel_fn,
        out_shape=jax.ShapeDtypeStruct((B, 100), jnp.float32),
        grid=(B // bt,),
        in_specs=[
            pl.BlockSpec((bt, 34, 102), lambda i: (i, 0, 0)),
            pl.BlockSpec((3, 102, 1024), lambda i: (0, 0, 0)),
            pl.BlockSpec((1, 1024), lambda i: (0, 0)),
            pl.BlockSpec((3, 576, 1024), lambda i: (0, 0, 0)),
            pl.BlockSpec((1, 1024), lambda i: (0, 0)),
            pl.BlockSpec((4096, 512), lambda i: (0, 0)),
            pl.BlockSpec((1, 512), lambda i: (0, 0)),
            pl.BlockSpec((512, 100), lambda i: (0, 0)),
            pl.BlockSpec((1, 100), lambda i: (0, 0)),
        ],
        out_specs=pl.BlockSpec((bt, 100), lambda i: (i, 0)),
        compiler_params=pltpu.CompilerParams(
            dimension_semantics=("parallel",),
            vmem_limit_bytes=64 * 1024 * 1024),
    )(xr, w1t, b1t, w2t, b2t, wf1, b_fc1, wf2, b_fc2)


# speedup vs baseline: 7.1183x; 1.0040x over previous
import functools

import jax
import jax.numpy as jnp
from jax.experimental import pallas as pl
from jax.experimental.pallas import tpu as pltpu


def _fused_cnn_kernel(bt, x_ref, w1t_ref, b1t_ref, w2t_ref, b2t_ref,
                      wf1_ref, bf1_ref, wf2_ref, bf2_ref, o_ref):
    f32 = jnp.float32
    x = x_ref[...]

    taps = [jnp.pad(x[:, dh:dh + 32, :], ((0, 0), (0, 0), (0, 26)))
            for dh in range(3)]
    lhs = jnp.concatenate(taps, axis=-1).reshape(bt * 32, 384)
    acc = jnp.dot(lhs, w1t_ref[...], preferred_element_type=f32)
    acc = jnp.maximum(acc[:, :512], acc[:, 512:])
    acc = acc.reshape(bt, 16, 2, 512).max(axis=2)
    acc = jnp.maximum(acc + b1t_ref[...], 0.0)
    a1 = acc.astype(jnp.bfloat16)

    zc = jnp.zeros((bt, 16, 32), jnp.bfloat16)
    s2 = jnp.concatenate([zc, a1, zc], axis=-1)
    zr = jnp.zeros((bt, 1, 576), jnp.bfloat16)
    s2 = jnp.concatenate([zr, s2, zr], axis=1)

    taps = [jnp.pad(s2[:, dh:dh + 16, :], ((0, 0), (0, 0), (0, 64)))
            for dh in range(3)]
    lhs = jnp.concatenate(taps, axis=-1).reshape(bt * 16, 1920)
    acc = jnp.dot(lhs, w2t_ref[...], preferred_element_type=f32)
    acc = jnp.maximum(acc[:, :512], acc[:, 512:])
    acc = acc.reshape(bt, 8, 2, 512).max(axis=2)
    acc = jnp.maximum(acc + b2t_ref[...], 0.0)
    c2 = acc.astype(jnp.bfloat16)

    xf = c2.reshape(bt, 4096)
    h1 = jnp.dot(xf, wf1_ref[...], preferred_element_type=f32)
    h1 = jnp.maximum(h1 + bf1_ref[...], 0.0).astype(jnp.bfloat16)
    z = jnp.dot(h1, wf2_ref[...], preferred_element_type=f32) + bf2_ref[...]
    z = z - jnp.max(z, axis=-1, keepdims=True)
    e = jnp.exp(z)
    o_ref[...] = (e / jnp.sum(e, axis=-1, keepdims=True)).astype(o_ref.dtype)


def kernel(x_nchw, w1, b1, w2, b2, w_fc1, b_fc1, w_fc2, b_fc2):
    B = x_nchw.shape[0]
    bt = 128 if B % 128 == 0 else B

    x = jnp.transpose(x_nchw, (0, 2, 3, 1))
    x = jnp.pad(x, ((0, 0), (1, 1), (1, 1), (0, 0)))
    xr = x.reshape(B, 34, 102).astype(jnp.bfloat16)

    w1r = w1.reshape(3, 3, 3, 32)
    E1 = (jnp.arange(34)[None, :, None, None]
          == 2 * jnp.arange(16)[None, None, None, :]
          + jnp.arange(2)[None, None, :, None]
          + jnp.arange(3)[:, None, None, None])
    w1t = jnp.einsum('kdpw,hkco->hdcpwo', E1.astype(w1.dtype), w1r)
    w1t = w1t.reshape(3, 102, 1024)
    w1t = jnp.pad(w1t, ((0, 0), (0, 26), (0, 0)))
    w1t = w1t.reshape(384, 1024).astype(jnp.bfloat16)

    w2r = w2.reshape(3, 3, 32, 64)
    E2 = (jnp.arange(18)[None, :, None, None]
          == 2 * jnp.arange(8)[None, None, None, :]
          + jnp.arange(2)[None, None, :, None]
          + jnp.arange(3)[:, None, None, None])
    w2t = jnp.einsum('kdpw,hkco->hdcpwo', E2.astype(w2.dtype), w2r)
    w2t = w2t.reshape(3, 576, 1024)
    w2t = jnp.pad(w2t, ((0, 0), (0, 64), (0, 0)))
    w2t = w2t.reshape(1920, 1024).astype(jnp.bfloat16)

    b1t = jnp.tile(b1, (1, 16))
    b2t = jnp.tile(b2, (1, 8))
    wf1 = w_fc1.astype(jnp.bfloat16)
    wf2 = w_fc2.astype(jnp.bfloat16)

    kernel_fn = functools.partial(_fused_cnn_kernel, bt)
    return pl.pallas_call(
        kernel_fn,
        out_shape=jax.ShapeDtypeStruct((B, 100), jnp.float32),
        grid=(B // bt,),
        in_specs=[
            pl.BlockSpec((bt, 34, 102), lambda i: (i, 0, 0)),
            pl.BlockSpec((384, 1024), lambda i: (0, 0)),
            pl.BlockSpec((1, 512), lambda i: (0, 0)),
            pl.BlockSpec((1920, 1024), lambda i: (0, 0)),
            pl.BlockSpec((1, 512), lambda i: (0, 0)),
            pl.BlockSpec((4096, 512), lambda i: (0, 0)),
            pl.BlockSpec((1, 512), lambda i: (0, 0)),
            pl.BlockSpec((512, 100), lambda i: (0, 0)),
            pl.BlockSpec((1, 100), lambda i: (0, 0)),
        ],
        out_specs=pl.BlockSpec((bt, 100), lambda i: (i, 0)),
        compiler_params=pltpu.CompilerParams(
            dimension_semantics=("parallel",),
            vmem_limit_bytes=64 * 1024 * 1024),
    )(xr, w1t, b1t, w2t, b2t, wf1, b_fc1, wf2, b_fc2)


# speedup vs baseline: 18.7345x; 2.6319x over previous
import functools

import jax
import jax.numpy as jnp
from jax.experimental import pallas as pl
from jax.experimental.pallas import tpu as pltpu


def _fused_cnn_kernel(bt, x_ref, w1t_ref, b1t_ref, w2t_ref, b2t_ref,
                      wf1_ref, bf1_ref, wf2_ref, bf2_ref, o_ref):
    f32 = jnp.float32
    x = x_ref[...]

    taps = [jnp.pad(x[dh:dh + 32], ((0, 0), (0, 0), (0, 26)))
            for dh in range(3)]
    lhs = jnp.concatenate(taps, axis=-1).reshape(32 * bt, 384)
    acc = jnp.dot(lhs, w1t_ref[...], preferred_element_type=f32)
    acc = jnp.maximum(acc[:, :512], acc[:, 512:])
    acc = acc.reshape(16, 2, bt, 512).max(axis=1)
    acc = jnp.maximum(acc + b1t_ref[...], 0.0)
    a1 = acc.astype(jnp.bfloat16)

    zc = jnp.zeros((16, bt, 32), jnp.bfloat16)
    s2 = jnp.concatenate([zc, a1, zc], axis=-1)
    zr = jnp.zeros((1, bt, 576), jnp.bfloat16)
    s2 = jnp.concatenate([zr, s2, zr], axis=0)

    taps = [jnp.pad(s2[dh:dh + 16], ((0, 0), (0, 0), (0, 64)))
            for dh in range(3)]
    lhs = jnp.concatenate(taps, axis=-1).reshape(16 * bt, 1920)
    acc = jnp.dot(lhs, w2t_ref[...], preferred_element_type=f32)
    acc = jnp.maximum(acc[:, :512], acc[:, 512:])
    acc = acc.reshape(8, 2, bt, 512).max(axis=1)
    acc = jnp.maximum(acc + b2t_ref[...], 0.0)
    c2 = acc.astype(jnp.bfloat16)

    xf = jnp.concatenate([c2[h] for h in range(8)], axis=-1)
    h1 = jnp.dot(xf, wf1_ref[...], preferred_element_type=f32)
    h1 = jnp.maximum(h1 + bf1_ref[...], 0.0).astype(jnp.bfloat16)
    z = jnp.dot(h1, wf2_ref[...], preferred_element_type=f32) + bf2_ref[...]
    z = z - jnp.max(z, axis=-1, keepdims=True)
    e = jnp.exp(z)
    o_ref[...] = (e / jnp.sum(e, axis=-1, keepdims=True)).astype(o_ref.dtype)


def kernel(x_nchw, w1, b1, w2, b2, w_fc1, b_fc1, w_fc2, b_fc2):
    B = x_nchw.shape[0]
    bt = 128 if B % 128 == 0 else B

    xp = jnp.pad(x_nchw, ((0, 0), (0, 0), (1, 1), (1, 1)))
    xt = jnp.transpose(xp, (2, 0, 3, 1))
    xr = xt.reshape(34, B, 102).astype(jnp.bfloat16)

    w1r = w1.reshape(3, 3, 3, 32)
    E1 = (jnp.arange(34)[None, :, None, None]
          == 2 * jnp.arange(16)[None, None, None, :]
          + jnp.arange(2)[None, None, :, None]
          + jnp.arange(3)[:, None, None, None])
    w1t = jnp.einsum('kdpw,hkco->hdcpwo', E1.astype(w1.dtype), w1r)
    w1t = jnp.pad(w1t.reshape(3, 102, 1024), ((0, 0), (0, 26), (0, 0)))
    w1t = w1t.reshape(384, 1024).astype(jnp.bfloat16)

    w2r = w2.reshape(3, 3, 32, 64)
    E2 = (jnp.arange(18)[None, :, None, None]
          == 2 * jnp.arange(8)[None, None, None, :]
          + jnp.arange(2)[None, None, :, None]
          + jnp.arange(3)[:, None, None, None])
    w2t = jnp.einsum('kdpw,hkco->hdcpwo', E2.astype(w2.dtype), w2r)
    w2t = jnp.pad(w2t.reshape(3, 576, 1024), ((0, 0), (0, 64), (0, 0)))
    w2t = w2t.reshape(1920, 1024).astype(jnp.bfloat16)

    b1t = jnp.tile(b1, (1, 16))
    b2t = jnp.tile(b2, (1, 8))
    wf1 = w_fc1.astype(jnp.bfloat16)
    wf2 = w_fc2.astype(jnp.bfloat16)

    kernel_fn = functools.partial(_fused_cnn_kernel, bt)
    return pl.pallas_call(
        kernel_fn,
        out_shape=jax.ShapeDtypeStruct((B, 100), jnp.float32),
        grid=(B // bt,),
        in_specs=[
            pl.BlockSpec((34, bt, 102), lambda i: (0, i, 0)),
            pl.BlockSpec((384, 1024), lambda i: (0, 0)),
            pl.BlockSpec((1, 512), lambda i: (0, 0)),
            pl.BlockSpec((1920, 1024), lambda i: (0, 0)),
            pl.BlockSpec((1, 512), lambda i: (0, 0)),
            pl.BlockSpec((4096, 512), lambda i: (0, 0)),
            pl.BlockSpec((1, 512), lambda i: (0, 0)),
            pl.BlockSpec((512, 100), lambda i: (0, 0)),
            pl.BlockSpec((1, 100), lambda i: (0, 0)),
        ],
        out_specs=pl.BlockSpec((bt, 100), lambda i: (i, 0)),
        compiler_params=pltpu.CompilerParams(
            dimension_semantics=("parallel",),
            vmem_limit_bytes=64 * 1024 * 1024),
    )(xr, w1t, b1t, w2t, b2t, wf1, b_fc1, wf2, b_fc2)


# speedup vs baseline: 27.0073x; 1.4416x over previous
import functools

import jax
import jax.numpy as jnp
from jax.experimental import pallas as pl
from jax.experimental.pallas import tpu as pltpu


def _fused_cnn_kernel(bt, x_ref, w1t_ref, b1t_ref, w2t_ref, b2t_ref,
                      wf1_ref, bf1_ref, wf2_ref, bf2_ref, o_ref):
    f32 = jnp.float32
    x = x_ref[...]

    taps = [jnp.pad(x[dh:dh + 32], ((0, 0), (0, 0), (0, 26)))
            for dh in range(3)]
    lhs = jnp.concatenate(taps, axis=-1).reshape(32 * bt, 384)
    acc = jnp.dot(lhs, w1t_ref[...], preferred_element_type=f32)
    a = acc.reshape(16, 2, bt, 1024)
    m = jnp.maximum(
        jnp.maximum(a[:, 0, :, :512], a[:, 0, :, 512:]),
        jnp.maximum(a[:, 1, :, :512], a[:, 1, :, 512:]))
    a1 = jnp.maximum(m + b1t_ref[...], 0.0).astype(jnp.bfloat16)

    zc = jnp.zeros((16, bt, 32), jnp.bfloat16)
    s2 = jnp.concatenate([zc, a1, zc], axis=-1)
    zr = jnp.zeros((1, bt, 576), jnp.bfloat16)
    s2 = jnp.concatenate([zr, s2, zr], axis=0)

    qs = []
    for q in range(4):
        taps = [jnp.pad(s2[dh:dh + 16, :, 128 * q:128 * q + 192],
                        ((0, 0), (0, 0), (0, 64))) for dh in range(3)]
        qs.append(jnp.concatenate(taps, axis=-1))
    lhs = jnp.stack(qs, axis=0).reshape(4 * 16 * bt, 768)
    acc = jnp.dot(lhs, w2t_ref[...], preferred_element_type=f32)
    a = acc.reshape(4, 8, 2, bt, 256)
    m = jnp.maximum(
        jnp.maximum(a[:, :, 0, :, :128], a[:, :, 0, :, 128:]),
        jnp.maximum(a[:, :, 1, :, :128], a[:, :, 1, :, 128:]))
    m = jnp.maximum(m + b2t_ref[...], 0.0)
    c2 = jnp.concatenate([m[q] for q in range(4)],
                         axis=-1).astype(jnp.bfloat16)

    xf = jnp.concatenate([c2[h] for h in range(8)], axis=-1)
    h1 = jnp.dot(xf, wf1_ref[...], preferred_element_type=f32)
    h1 = jnp.maximum(h1 + bf1_ref[...], 0.0).astype(jnp.bfloat16)
    z = jnp.dot(h1, wf2_ref[...], preferred_element_type=f32) + bf2_ref[...]
    z = z - jnp.max(z, axis=-1, keepdims=True)
    e = jnp.exp(z)
    o_ref[...] = (e / jnp.sum(e, axis=-1, keepdims=True)).astype(o_ref.dtype)


def kernel(x_nchw, w1, b1, w2, b2, w_fc1, b_fc1, w_fc2, b_fc2):
    B = x_nchw.shape[0]
    bt = 128 if B % 128 == 0 else B

    xp = jnp.pad(x_nchw, ((0, 0), (0, 0), (1, 1), (1, 1)))
    xt = jnp.transpose(xp, (2, 0, 3, 1))
    xr = xt.reshape(34, B, 102).astype(jnp.bfloat16)

    w1r = w1.reshape(3, 3, 3, 32)
    E1 = (jnp.arange(34)[None, :, None, None]
          == 2 * jnp.arange(16)[None, None, None, :]
          + jnp.arange(2)[None, None, :, None]
          + jnp.arange(3)[:, None, None, None])
    w1t = jnp.einsum('kdpw,hkco->hdcpwo', E1.astype(w1.dtype), w1r)
    w1t = jnp.pad(w1t.reshape(3, 102, 1024), ((0, 0), (0, 26), (0, 0)))
    w1t = w1t.reshape(384, 1024).astype(jnp.bfloat16)

    w2r = w2.reshape(3, 3, 32, 64)
    E2 = (jnp.arange(6)[None, :, None, None]
          == 2 * jnp.arange(2)[None, None, None, :]
          + jnp.arange(2)[None, None, :, None]
          + jnp.arange(3)[:, None, None, None])
    w2t = jnp.einsum('kdpw,hkco->hdcpwo', E2.astype(w2.dtype), w2r)
    w2t = jnp.pad(w2t.reshape(3, 192, 256), ((0, 0), (0, 64), (0, 0)))
    w2t = w2t.reshape(768, 256).astype(jnp.bfloat16)

    b1t = jnp.tile(b1, (1, 16))
    b2t = jnp.tile(b2, (1, 2))
    wf1 = w_fc1.astype(jnp.bfloat16)
    wf2 = w_fc2.astype(jnp.bfloat16)

    kernel_fn = functools.partial(_fused_cnn_kernel, bt)
    return pl.pallas_call(
        kernel_fn,
        out_shape=jax.ShapeDtypeStruct((B, 100), jnp.float32),
        grid=(B // bt,),
        in_specs=[
            pl.BlockSpec((34, bt, 102), lambda i: (0, i, 0)),
            pl.BlockSpec((384, 1024), lambda i: (0, 0)),
            pl.BlockSpec((1, 512), lambda i: (0, 0)),
            pl.BlockSpec((768, 256), lambda i: (0, 0)),
            pl.BlockSpec((1, 128), lambda i: (0, 0)),
            pl.BlockSpec((4096, 512), lambda i: (0, 0)),
            pl.BlockSpec((1, 512), lambda i: (0, 0)),
            pl.BlockSpec((512, 100), lambda i: (0, 0)),
            pl.BlockSpec((1, 100), lambda i: (0, 0)),
        ],
        out_specs=pl.BlockSpec((bt, 100), lambda i: (i, 0)),
        compiler_params=pltpu.CompilerParams(
            dimension_semantics=("parallel",),
            vmem_limit_bytes=64 * 1024 * 1024),
    )(xr, w1t, b1t, w2t, b2t, wf1, b_fc1, wf2, b_fc2)


# speedup vs baseline: 29.4787x; 1.0915x over previous
import functools

import jax
import jax.numpy as jnp
from jax.experimental import pallas as pl
from jax.experimental.pallas import tpu as pltpu


def _fused_cnn_kernel(bt, x_ref, w1t_ref, b1t_ref, w2t_ref, b2t_ref,
                      wf1_ref, bf1_ref, wf2_ref, bf2_ref, o_ref):
    f32 = jnp.float32

    zx = jnp.zeros((1, bt, 128), jnp.bfloat16)
    xh = jnp.concatenate(
        [zx, jnp.pad(x_ref[...], ((0, 0), (0, 0), (0, 32))), zx], axis=0)

    taps = [xh[dh:dh + 32] for dh in range(3)]
    lhs = jnp.concatenate(taps, axis=-1).reshape(32 * bt, 384)
    acc = jnp.dot(lhs, w1t_ref[...], preferred_element_type=f32)
    a = acc.reshape(16, 2, bt, 1024)
    m = jnp.maximum(
        jnp.maximum(a[:, 0, :, :512], a[:, 0, :, 512:]),
        jnp.maximum(a[:, 1, :, :512], a[:, 1, :, 512:]))
    a1 = jnp.maximum(m + b1t_ref[...], 0.0).astype(jnp.bfloat16)

    zc = jnp.zeros((16, bt, 32), jnp.bfloat16)
    s2 = jnp.concatenate([zc, a1, zc], axis=-1)
    zr = jnp.zeros((1, bt, 576), jnp.bfloat16)
    s2 = jnp.concatenate([zr, s2, zr], axis=0)

    qs = []
    for q in range(4):
        taps = [jnp.pad(s2[dh:dh + 16, :, 128 * q:128 * q + 192],
                        ((0, 0), (0, 0), (0, 64))) for dh in range(3)]
        qs.append(jnp.concatenate(taps, axis=-1))
    lhs = jnp.stack(qs, axis=0).reshape(4 * 16 * bt, 768)
    acc = jnp.dot(lhs, w2t_ref[...], preferred_element_type=f32)
    a = acc.reshape(4, 8, 2, bt, 256)
    m = jnp.maximum(
        jnp.maximum(a[:, :, 0, :, :128], a[:, :, 0, :, 128:]),
        jnp.maximum(a[:, :, 1, :, :128], a[:, :, 1, :, 128:]))
    m = jnp.maximum(m + b2t_ref[...], 0.0)
    c2 = jnp.concatenate([m[q] for q in range(4)],
                         axis=-1).astype(jnp.bfloat16)

    xf = jnp.concatenate([c2[h] for h in range(8)], axis=-1)
    h1 = jnp.dot(xf, wf1_ref[...], preferred_element_type=f32)
    h1 = jnp.maximum(h1 + bf1_ref[...], 0.0).astype(jnp.bfloat16)
    z = jnp.dot(h1, wf2_ref[...], preferred_element_type=f32) + bf2_ref[...]
    z = z - jnp.max(z, axis=-1, keepdims=True)
    e = jnp.exp(z)
    o_ref[...] = (e / jnp.sum(e, axis=-1, keepdims=True)).astype(o_ref.dtype)


def kernel(x_nchw, w1, b1, w2, b2, w_fc1, b_fc1, w_fc2, b_fc2):
    B = x_nchw.shape[0]
    bt = 128 if B % 128 == 0 else B

    xt = jnp.transpose(x_nchw, (2, 0, 3, 1))
    xr = xt.reshape(32, B, 96).astype(jnp.bfloat16)

    w1r = w1.reshape(3, 3, 3, 32)
    E1 = (jnp.arange(32)[None, :, None, None] + 1
          == 2 * jnp.arange(16)[None, None, None, :]
          + jnp.arange(2)[None, None, :, None]
          + jnp.arange(3)[:, None, None, None])
    w1t = jnp.einsum('kdpw,hkco->hdcpwo', E1.astype(w1.dtype), w1r)
    w1t = jnp.pad(w1t.reshape(3, 96, 1024), ((0, 0), (0, 32), (0, 0)))
    w1t = w1t.reshape(384, 1024).astype(jnp.bfloat16)

    w2r = w2.reshape(3, 3, 32, 64)
    E2 = (jnp.arange(6)[None, :, None, None]
          == 2 * jnp.arange(2)[None, None, None, :]
          + jnp.arange(2)[None, None, :, None]
          + jnp.arange(3)[:, None, None, None])
    w2t = jnp.einsum('kdpw,hkco->hdcpwo', E2.astype(w2.dtype), w2r)
    w2t = jnp.pad(w2t.reshape(3, 192, 256), ((0, 0), (0, 64), (0, 0)))
    w2t = w2t.reshape(768, 256).astype(jnp.bfloat16)

    b1t = jnp.tile(b1, (1, 16))
    b2t = jnp.tile(b2, (1, 2))
    wf1 = w_fc1.astype(jnp.bfloat16)
    wf2 = w_fc2.astype(jnp.bfloat16)

    kernel_fn = functools.partial(_fused_cnn_kernel, bt)
    return pl.pallas_call(
        kernel_fn,
        out_shape=jax.ShapeDtypeStruct((B, 100), jnp.float32),
        grid=(B // bt,),
        in_specs=[
            pl.BlockSpec((32, bt, 96), lambda i: (0, i, 0)),
            pl.BlockSpec((384, 1024), lambda i: (0, 0)),
            pl.BlockSpec((1, 512), lambda i: (0, 0)),
            pl.BlockSpec((768, 256), lambda i: (0, 0)),
            pl.BlockSpec((1, 128), lambda i: (0, 0)),
            pl.BlockSpec((4096, 512), lambda i: (0, 0)),
            pl.BlockSpec((1, 512), lambda i: (0, 0)),
            pl.BlockSpec((512, 100), lambda i: (0, 0)),
            pl.BlockSpec((1, 100), lambda i: (0, 0)),
        ],
        out_specs=pl.BlockSpec((bt, 100), lambda i: (i, 0)),
        compiler_params=pltpu.CompilerParams(
            dimension_semantics=("parallel",),
            vmem_limit_bytes=64 * 1024 * 1024),
    )(xr, w1t, b1t, w2t, b2t, wf1, b_fc1, wf2, b_fc2)


# speedup vs baseline: 31.2831x; 1.0612x over previous
import functools

import jax
import jax.numpy as jnp
from jax.experimental import pallas as pl
from jax.experimental.pallas import tpu as pltpu


def _fused_cnn_kernel(bt, x_ref, w1t_ref, b1t_ref, w2t_ref, b2t_ref,
                      wf1_ref, bf1_ref, wf2_ref, bf2_ref, o_ref):
    f32 = jnp.float32

    zx = jnp.zeros((1, bt, 128), jnp.bfloat16)
    xh = jnp.concatenate(
        [zx, jnp.pad(x_ref[...], ((0, 0), (0, 0), (0, 32))), zx], axis=0)

    taps = [xh[dh:dh + 32] for dh in range(3)]
    lhs = jnp.concatenate(taps, axis=-1).reshape(32 * bt, 384)
    acc = jnp.dot(lhs, w1t_ref[...], preferred_element_type=f32)
    a = acc.reshape(16, 2, bt, 1024)
    m = jnp.maximum(
        jnp.maximum(a[:, 0, :, :512], a[:, 0, :, 512:]),
        jnp.maximum(a[:, 1, :, :512], a[:, 1, :, 512:]))
    a1 = jnp.maximum(m + b1t_ref[...], 0.0).astype(jnp.bfloat16)

    zc = jnp.zeros((16, bt, 32), jnp.bfloat16)
    s2 = jnp.concatenate([zc, a1, zc], axis=-1)
    zr = jnp.zeros((1, bt, 576), jnp.bfloat16)
    s2 = jnp.concatenate([zr, s2, zr], axis=0)

    qs = []
    for q in range(4):
        taps = [jnp.pad(s2[dh:dh + 16, :, 128 * q:128 * q + 192],
                        ((0, 0), (0, 0), (0, 64))) for dh in range(3)]
        qs.append(jnp.concatenate(taps, axis=-1))
    lhs = jnp.stack(qs, axis=0).reshape(4 * 16 * bt, 768)
    acc = jnp.dot(lhs, w2t_ref[...], preferred_element_type=f32)
    a = acc.reshape(4, 8, 2, bt, 256)
    m = jnp.maximum(
        jnp.maximum(a[:, :, 0, :, :128], a[:, :, 0, :, 128:]),
        jnp.maximum(a[:, :, 1, :, :128], a[:, :, 1, :, 128:]))
    m = jnp.maximum(m + b2t_ref[...], 0.0)
    c2 = jnp.concatenate([m[q] for q in range(4)],
                         axis=-1).astype(jnp.bfloat16)

    xf = jnp.concatenate([c2[h] for h in range(8)], axis=-1)
    h1 = jnp.dot(xf, wf1_ref[...], preferred_element_type=f32)
    h1 = jnp.maximum(h1 + bf1_ref[...], 0.0).astype(jnp.bfloat16)
    z = jnp.dot(h1, wf2_ref[...], preferred_element_type=f32) + bf2_ref[...]
    z = z - jnp.max(z, axis=-1, keepdims=True)
    e = jnp.exp(z)
    o_ref[...] = (e / jnp.sum(e, axis=-1, keepdims=True)).astype(o_ref.dtype)


def kernel(x_nchw, w1, b1, w2, b2, w_fc1, b_fc1, w_fc2, b_fc2):
    B = x_nchw.shape[0]
    bt = 128 if B % 128 == 0 else B

    xt = jnp.transpose(x_nchw, (2, 0, 1, 3))
    xr = xt.reshape(32, B, 96).astype(jnp.bfloat16)

    w1r = w1.reshape(3, 3, 3, 32)
    E1 = (jnp.arange(32)[None, :, None, None] + 1
          == 2 * jnp.arange(16)[None, None, None, :]
          + jnp.arange(2)[None, None, :, None]
          + jnp.arange(3)[:, None, None, None])
    w1t = jnp.einsum('kdpw,hkco->hcdpwo', E1.astype(w1.dtype), w1r)
    w1t = jnp.pad(w1t.reshape(3, 96, 1024), ((0, 0), (0, 32), (0, 0)))
    w1t = w1t.reshape(384, 1024).astype(jnp.bfloat16)

    w2r = w2.reshape(3, 3, 32, 64)
    E2 = (jnp.arange(6)[None, :, None, None]
          == 2 * jnp.arange(2)[None, None, None, :]
          + jnp.arange(2)[None, None, :, None]
          + jnp.arange(3)[:, None, None, None])
    w2t = jnp.einsum('kdpw,hkco->hdcpwo', E2.astype(w2.dtype), w2r)
    w2t = jnp.pad(w2t.reshape(3, 192, 256), ((0, 0), (0, 64), (0, 0)))
    w2t = w2t.reshape(768, 256).astype(jnp.bfloat16)

    b1t = jnp.tile(b1, (1, 16))
    b2t = jnp.tile(b2, (1, 2))
    wf1 = w_fc1.astype(jnp.bfloat16)
    wf2 = w_fc2.astype(jnp.bfloat16)

    kernel_fn = functools.partial(_fused_cnn_kernel, bt)
    return pl.pallas_call(
        kernel_fn,
        out_shape=jax.ShapeDtypeStruct((B, 100), jnp.float32),
        grid=(B // bt,),
        in_specs=[
            pl.BlockSpec((32, bt, 96), lambda i: (0, i, 0)),
            pl.BlockSpec((384, 1024), lambda i: (0, 0)),
            pl.BlockSpec((1, 512), lambda i: (0, 0)),
            pl.BlockSpec((768, 256), lambda i: (0, 0)),
            pl.BlockSpec((1, 128), lambda i: (0, 0)),
            pl.BlockSpec((4096, 512), lambda i: (0, 0)),
            pl.BlockSpec((1, 512), lambda i: (0, 0)),
            pl.BlockSpec((512, 100), lambda i: (0, 0)),
            pl.BlockSpec((1, 100), lambda i: (0, 0)),
        ],
        out_specs=pl.BlockSpec((bt, 100), lambda i: (i, 0)),
        compiler_params=pltpu.CompilerParams(
            dimension_semantics=("parallel",),
            vmem_limit_bytes=64 * 1024 * 1024),
    )(xr, w1t, b1t, w2t, b2t, wf1, b_fc1, wf2, b_fc2)


# speedup vs baseline: 33.1224x; 1.0588x over previous
import functools

import jax
import jax.numpy as jnp
from jax.experimental import pallas as pl
from jax.experimental.pallas import tpu as pltpu


def _fused_cnn_kernel(bt, x_ref, w1t_ref, b1t_ref, w2t_ref, b2t_ref,
                      wf1_ref, bf1_ref, wf2_ref, bf2_ref, o_ref):
    f32 = jnp.float32

    zx = jnp.zeros((1, bt, 128), jnp.bfloat16)
    xh = jnp.concatenate(
        [zx, jnp.pad(x_ref[...], ((0, 0), (0, 0), (0, 32))), zx], axis=0)

    a1s = []
    for hc in range(2):
        taps = [xh[16 * hc + dh:16 * hc + dh + 16] for dh in range(3)]
        lhs = jnp.concatenate(taps, axis=-1).reshape(16 * bt, 384)
        acc = jnp.dot(lhs, w1t_ref[...], preferred_element_type=f32)
        a = acc.reshape(8, 2, bt, 1024)
        m = jnp.maximum(
            jnp.maximum(a[:, 0, :, :512], a[:, 0, :, 512:]),
            jnp.maximum(a[:, 1, :, :512], a[:, 1, :, 512:]))
        a1s.append(jnp.maximum(m + b1t_ref[...], 0.0).astype(jnp.bfloat16))
    a1 = jnp.concatenate(a1s, axis=0)

    zc = jnp.zeros((16, bt, 32), jnp.bfloat16)
    s2 = jnp.concatenate([zc, a1, zc], axis=-1)
    zr = jnp.zeros((1, bt, 576), jnp.bfloat16)
    s2 = jnp.concatenate([zr, s2, zr], axis=0)

    c2s = []
    for hc in range(2):
        qs = []
        for q in range(4):
            taps = [jnp.pad(
                s2[8 * hc + dh:8 * hc + dh + 8, :, 128 * q:128 * q + 192],
                ((0, 0), (0, 0), (0, 64))) for dh in range(3)]
            qs.append(jnp.concatenate(taps, axis=-1))
        lhs = jnp.stack(qs, axis=0).reshape(4 * 8 * bt, 768)
        acc = jnp.dot(lhs, w2t_ref[...], preferred_element_type=f32)
        a = acc.reshape(4, 4, 2, bt, 256)
        m = jnp.maximum(
            jnp.maximum(a[:, :, 0, :, :128], a[:, :, 0, :, 128:]),
            jnp.maximum(a[:, :, 1, :, :128], a[:, :, 1, :, 128:]))
        m = jnp.maximum(m + b2t_ref[...], 0.0)
        c2s.append(jnp.concatenate([m[q] for q in range(4)],
                                   axis=-1).astype(jnp.bfloat16))
    c2 = jnp.concatenate(c2s, axis=0)

    xf = jnp.concatenate([c2[h] for h in range(8)], axis=-1)
    h1 = jnp.dot(xf, wf1_ref[...], preferred_element_type=f32)
    h1 = jnp.maximum(h1 + bf1_ref[...], 0.0).astype(jnp.bfloat16)
    z = jnp.dot(h1, wf2_ref[...], preferred_element_type=f32) + bf2_ref[...]
    z = z - jnp.max(z, axis=-1, keepdims=True)
    e = jnp.exp(z)
    o_ref[...] = (e / jnp.sum(e, axis=-1, keepdims=True)).astype(o_ref.dtype)


def kernel(x_nchw, w1, b1, w2, b2, w_fc1, b_fc1, w_fc2, b_fc2):
    B = x_nchw.shape[0]
    bt = 256 if B % 256 == 0 else B

    xt = jnp.transpose(x_nchw, (2, 0, 1, 3))
    xr = xt.reshape(32, B, 96).astype(jnp.bfloat16)

    w1r = w1.reshape(3, 3, 3, 32)
    E1 = (jnp.arange(32)[None, :, None, None] + 1
          == 2 * jnp.arange(16)[None, None, None, :]
          + jnp.arange(2)[None, None, :, None]
          + jnp.arange(3)[:, None, None, None])
    w1t = jnp.einsum('kdpw,hkco->hcdpwo', E1.astype(w1.dtype), w1r)
    w1t = jnp.pad(w1t.reshape(3, 96, 1024), ((0, 0), (0, 32), (0, 0)))
    w1t = w1t.reshape(384, 1024).astype(jnp.bfloat16)

    w2r = w2.reshape(3, 3, 32, 64)
    E2 = (jnp.arange(6)[None, :, None, None]
          == 2 * jnp.arange(2)[None, None, None, :]
          + jnp.arange(2)[None, None, :, None]
          + jnp.arange(3)[:, None, None, None])
    w2t = jnp.einsum('kdpw,hkco->hdcpwo', E2.astype(w2.dtype), w2r)
    w2t = jnp.pad(w2t.reshape(3, 192, 256), ((0, 0), (0, 64), (0, 0)))
    w2t = w2t.reshape(768, 256).astype(jnp.bfloat16)

    b1t = jnp.tile(b1, (1, 16))
    b2t = jnp.tile(b2, (1, 2))
    wf1 = w_fc1.astype(jnp.bfloat16)
    wf2 = w_fc2.astype(jnp.bfloat16)

    kernel_fn = functools.partial(_fused_cnn_kernel, bt)
    return pl.pallas_call(
        kernel_fn,
        out_shape=jax.ShapeDtypeStruct((B, 100), jnp.float32),
        grid=(B // bt,),
        in_specs=[
            pl.BlockSpec((32, bt, 96), lambda i: (0, i, 0)),
            pl.BlockSpec((384, 1024), lambda i: (0, 0)),
            pl.BlockSpec((1, 512), lambda i: (0, 0)),
            pl.BlockSpec((768, 256), lambda i: (0, 0)),
            pl.BlockSpec((1, 128), lambda i: (0, 0)),
            pl.BlockSpec((4096, 512), lambda i: (0, 0)),
            pl.BlockSpec((1, 512), lambda i: (0, 0)),
            pl.BlockSpec((512, 100), lambda i: (0, 0)),
            pl.BlockSpec((1, 100), lambda i: (0, 0)),
        ],
        out_specs=pl.BlockSpec((bt, 100), lambda i: (i, 0)),
        compiler_params=pltpu.CompilerParams(
            dimension_semantics=("parallel",),
            vmem_limit_bytes=64 * 1024 * 1024),
    )(xr, w1t, b1t, w2t, b2t, wf1, b_fc1, wf2, b_fc2)
